# Initial kernel scaffold; baseline (speedup 1.0000x reference)
import jax
import jax.numpy as jnp
from jax.experimental import pallas as pl


def kernel(x, edge_index, W1, V1, b1, W2, V2, b2):
    raise NotImplementedError("write your pallas kernel here")


# speedup vs baseline: 23.4296x; 23.4296x over previous
import functools

import jax
import jax.numpy as jnp
from jax import lax
from jax.experimental import pallas as pl
from jax.experimental.pallas import tpu as pltpu
from jax.experimental.pallas import tpu_sc as plsc

_N = 10000
_E = 320000
_F = 128
_H = 32
_C = 10
_CP = 16

_NP = 10240
_NC = 2
_NS = 16
_NW = _NC * _NS
_EW = _E // _NW
_K = 80
_CH = _EW // _K
_RPT = _NP // _NS
_DEGW = 8


def _sc_mesh():
    return plsc.VectorSubcoreMesh(
        core_axis_name="c", subcore_axis_name="s",
        num_cores=_NC, num_subcores=_NS)


def _sc_histogram(dst3, ones, zeros):
    @functools.partial(
        pl.kernel,
        out_type=jax.ShapeDtypeStruct((_NC * _NP, _DEGW), jnp.float32),
        mesh=_sc_mesh(),
        compiler_params=pltpu.CompilerParams(use_tc_tiling_on_sc=False),
        scratch_types=[
            pltpu.VMEM((_CH, _K), jnp.int32),
            pltpu.VMEM((_K, _DEGW), jnp.float32),
            pltpu.VMEM_SHARED((_NP, _DEGW), jnp.float32),
        ],
    )
    def histk(dst_hbm, ones_hbm, zeros_hbm, out_hbm, idx_v, ones_v, deg_sh):
        cid = lax.axis_index("c")
        sid = lax.axis_index("s")
        wid = cid * _NS + sid
        r0 = sid * _RPT
        pltpu.sync_copy(zeros_hbm.at[pl.ds(r0, _RPT)], deg_sh.at[pl.ds(r0, _RPT)])
        pltpu.sync_copy(dst_hbm.at[wid], idx_v)
        pltpu.sync_copy(ones_hbm, ones_v)
        plsc.subcore_barrier()

        def body(j, carry):
            pltpu.sync_copy(ones_v, deg_sh.at[idx_v.at[j]], add=True)
            return carry

        lax.fori_loop(0, _CH, body, 0)
        plsc.subcore_barrier()
        pltpu.sync_copy(deg_sh.at[pl.ds(r0, _RPT)],
                        out_hbm.at[pl.ds(cid * _NP + r0, _RPT)])

    return histk(dst3, ones, zeros)


def _sc_prop(g, src3, dst3, zeros, d):
    @functools.partial(
        pl.kernel,
        out_type=jax.ShapeDtypeStruct((_NC * _NP, d), jnp.float32),
        mesh=_sc_mesh(),
        compiler_params=pltpu.CompilerParams(use_tc_tiling_on_sc=False),
        scratch_types=[
            pltpu.VMEM((_CH, _K), jnp.int32),
            pltpu.VMEM((_CH, _K), jnp.int32),
            pltpu.VMEM((_K, d), jnp.float32),
            pltpu.VMEM_SHARED((_NP, d), jnp.float32),
            pltpu.SemaphoreType.DMA,
        ],
    )
    def propk(g_hbm, src_hbm, dst_hbm, zeros_hbm, out_hbm,
              src_v, dst_v, rows_v, agg_sh, sem):
        cid = lax.axis_index("c")
        sid = lax.axis_index("s")
        wid = cid * _NS + sid
        r0 = sid * _RPT
        pltpu.sync_copy(zeros_hbm.at[pl.ds(r0, _RPT)], agg_sh.at[pl.ds(r0, _RPT)])
        pltpu.sync_copy(src_hbm.at[wid], src_v)
        pltpu.sync_copy(dst_hbm.at[wid], dst_v)
        plsc.subcore_barrier()

        def body(j, carry):
            pltpu.async_copy(g_hbm.at[src_v.at[j]], rows_v, sem).wait()
            pltpu.sync_copy(rows_v, agg_sh.at[dst_v.at[j]], add=True)
            return carry

        lax.fori_loop(0, _CH, body, 0)
        plsc.subcore_barrier()
        pltpu.sync_copy(agg_sh.at[pl.ds(r0, _RPT)],
                        out_hbm.at[pl.ds(cid * _NP + r0, _RPT)])

    return propk(g, src3, dst3, zeros)


def _tc_phase1(x, w1, v1, degp):
    def body(x_ref, w_ref, v_ref, degp_ref, g_ref, xv_ref, dis_ref):
        deg = degp_ref[0:_N, 0:1] + degp_ref[_NP:_NP + _N, 0:1]
        dis = jnp.where(deg > 0, 1.0 / jnp.sqrt(jnp.maximum(deg, 1e-12)), 0.0)
        xv = x_ref[...]
        g_ref[...] = dis * jnp.dot(xv, w_ref[...], preferred_element_type=jnp.float32)
        xv_ref[...] = jnp.dot(xv, v_ref[...], preferred_element_type=jnp.float32)
        dis_ref[...] = dis

    return pl.pallas_call(
        body,
        out_shape=(
            jax.ShapeDtypeStruct((_N, _H), jnp.float32),
            jax.ShapeDtypeStruct((_N, _H), jnp.float32),
            jax.ShapeDtypeStruct((_N, 1), jnp.float32),
        ),
    )(x, w1, v1, degp)


def _tc_phase2(s1p, xv1, dis, b1, w2p, v2p):
    def body(s_ref, xv_ref, dis_ref, b_ref, w_ref, v_ref, g_ref, xv2_ref):
        dis = dis_ref[...]
        s = s_ref[0:_N] + s_ref[_NP:_NP + _N]
        x1 = jnp.maximum(dis * s + xv_ref[...] + b_ref[...], 0.0)
        g_ref[...] = dis * jnp.dot(x1, w_ref[...], preferred_element_type=jnp.float32)
        xv2_ref[...] = jnp.dot(x1, v_ref[...], preferred_element_type=jnp.float32)

    return pl.pallas_call(
        body,
        out_shape=(
            jax.ShapeDtypeStruct((_N, _CP), jnp.float32),
            jax.ShapeDtypeStruct((_N, _CP), jnp.float32),
        ),
    )(s1p, xv1, dis, b1, w2p, v2p)


def _tc_phase3(s2p, xv2, dis, b2p):
    def body(s_ref, xv_ref, dis_ref, b_ref, out_ref):
        dis = dis_ref[...]
        z = jnp.maximum(dis * (s_ref[0:_N] + s_ref[_NP:_NP + _N])
                        + xv_ref[...] + b_ref[...], 0.0)
        col = lax.broadcasted_iota(jnp.int32, (_N, _CP), 1)
        valid = col < _C
        zm = jnp.where(valid, z, -jnp.inf)
        m = jnp.max(zm, axis=1, keepdims=True)
        e = jnp.where(valid, jnp.exp(z - m), 0.0)
        lse = jnp.log(jnp.sum(e, axis=1, keepdims=True))
        out_ref[...] = z - m - lse

    return pl.pallas_call(
        body,
        out_shape=jax.ShapeDtypeStruct((_N, _CP), jnp.float32),
    )(s2p, xv2, dis, b2p)


def kernel(x, edge_index, W1, V1, b1, W2, V2, b2):
    src3 = edge_index[0].reshape(_NW, _CH, _K)
    dst3 = edge_index[1].reshape(_NW, _CH, _K)
    ones = jnp.ones((_K, _DEGW), jnp.float32)
    zeros_d = jnp.zeros((_NP, _DEGW), jnp.float32)
    zeros_h = jnp.zeros((_NP, _H), jnp.float32)
    zeros_c = jnp.zeros((_NP, _CP), jnp.float32)

    degp = _sc_histogram(dst3, ones, zeros_d)
    g1, xv1, dis = _tc_phase1(x, W1, V1, degp)
    s1p = _sc_prop(g1, src3, dst3, zeros_h, _H)
    w2p = jnp.pad(W2, ((0, 0), (0, _CP - _C)))
    v2p = jnp.pad(V2, ((0, 0), (0, _CP - _C)))
    g2, xv2 = _tc_phase2(s1p, xv1, dis, b1.reshape(1, _H), w2p, v2p)
    s2p = _sc_prop(g2, src3, dst3, zeros_c, _CP)
    out16 = _tc_phase3(s2p, xv2, dis, jnp.pad(b2, (0, _CP - _C)).reshape(1, _CP))
    return out16[:, :_C]


# speedup vs baseline: 35.5505x; 1.5173x over previous
import functools

import jax
import jax.numpy as jnp
from jax import lax
from jax.experimental import pallas as pl
from jax.experimental.pallas import tpu as pltpu
from jax.experimental.pallas import tpu_sc as plsc

_N = 10000
_E = 320000
_F = 128
_H = 32
_C = 10
_CP = 16

_NP = 10240
_NC = 2
_NS = 16
_NW = _NC * _NS
_EW = _E // _NW
_K = 80
_CH = _EW // _K
_RPT = _NP // _NS
_DEGW = 8
_NB = 5


def _sc_mesh():
    return plsc.VectorSubcoreMesh(
        core_axis_name="c", subcore_axis_name="s",
        num_cores=_NC, num_subcores=_NS)


def _sc_histogram(dst3, ones, zeros):
    @functools.partial(
        pl.kernel,
        out_type=jax.ShapeDtypeStruct((_NC * _NP, _DEGW), jnp.float32),
        mesh=_sc_mesh(),
        compiler_params=pltpu.CompilerParams(use_tc_tiling_on_sc=False),
        scratch_types=[
            pltpu.VMEM((_CH, _K), jnp.int32),
            pltpu.VMEM((_K, _DEGW), jnp.float32),
            pltpu.VMEM_SHARED((_NP, _DEGW), jnp.float32),
        ] + [pltpu.SemaphoreType.DMA] * _NB,
    )
    def histk(dst_hbm, ones_hbm, zeros_hbm, out_hbm, idx_v, ones_v, deg_sh,
              *sems):
        cid = lax.axis_index("c")
        sid = lax.axis_index("s")
        wid = cid * _NS + sid
        r0 = sid * _RPT
        pltpu.sync_copy(zeros_hbm.at[pl.ds(r0, _RPT)], deg_sh.at[pl.ds(r0, _RPT)])
        pltpu.sync_copy(dst_hbm.at[wid], idx_v)
        pltpu.sync_copy(ones_hbm, ones_v)
        plsc.subcore_barrier()

        def start_s(jj, b):
            pltpu.async_copy(ones_v, deg_sh.at[idx_v.at[jj]], sems[b], add=True)

        def wait_s(jj, b):
            pltpu.make_async_copy(ones_v, deg_sh.at[idx_v.at[jj]], sems[b]).wait()

        for b in range(_NB):
            start_s(b, b)

        @pl.loop(1, _CH // _NB)
        def _grp(g):
            for b in range(_NB):
                wait_s((g - 1) * _NB + b, b)
                start_s(g * _NB + b, b)

        for b in range(_NB):
            wait_s(_CH - _NB + b, b)
        plsc.subcore_barrier()
        pltpu.sync_copy(deg_sh.at[pl.ds(r0, _RPT)],
                        out_hbm.at[pl.ds(cid * _NP + r0, _RPT)])

    return histk(dst3, ones, zeros)


def _sc_prop(g, src3, dst3, zeros, d):
    @functools.partial(
        pl.kernel,
        out_type=jax.ShapeDtypeStruct((_NC * _NP, d), jnp.float32),
        mesh=_sc_mesh(),
        compiler_params=pltpu.CompilerParams(use_tc_tiling_on_sc=False),
        scratch_types=[
            pltpu.VMEM((_CH, _K), jnp.int32),
            pltpu.VMEM((_CH, _K), jnp.int32),
            pltpu.VMEM((_NB, _K, d), jnp.float32),
            pltpu.VMEM_SHARED((_NP, d), jnp.float32),
        ] + [pltpu.SemaphoreType.DMA] * (2 * _NB),
    )
    def propk(g_hbm, src_hbm, dst_hbm, zeros_hbm, out_hbm,
              src_v, dst_v, rows_v, agg_sh, *sems):
        sem_g = sems[:_NB]
        sem_s = sems[_NB:]
        cid = lax.axis_index("c")
        sid = lax.axis_index("s")
        wid = cid * _NS + sid
        r0 = sid * _RPT
        pltpu.sync_copy(zeros_hbm.at[pl.ds(r0, _RPT)], agg_sh.at[pl.ds(r0, _RPT)])
        pltpu.sync_copy(src_hbm.at[wid], src_v)
        pltpu.sync_copy(dst_hbm.at[wid], dst_v)
        plsc.subcore_barrier()

        def start_g(jj, b):
            pltpu.async_copy(g_hbm.at[src_v.at[jj]], rows_v.at[b], sem_g[b])

        def wait_g(jj, b):
            pltpu.make_async_copy(g_hbm.at[src_v.at[jj]], rows_v.at[b],
                                  sem_g[b]).wait()

        def start_s(jj, b):
            pltpu.async_copy(rows_v.at[b], agg_sh.at[dst_v.at[jj]], sem_s[b],
                             add=True)

        def wait_s(jj, b):
            pltpu.make_async_copy(rows_v.at[b], agg_sh.at[dst_v.at[jj]],
                                  sem_s[b]).wait()

        start_g(0, 0)
        for jj in range(_NB - 1):
            b, b1 = jj % _NB, (jj + 1) % _NB
            wait_g(jj, b)
            start_s(jj, b)
            start_g(jj + 1, b1)

        @pl.loop(0, (_CH - _NB) // _NB)
        def _grp(grp):
            for k in range(_NB):
                jj = (_NB - 1) + grp * _NB + k
                b = (_NB - 1 + k) % _NB
                b1 = (b + 1) % _NB
                wait_s(jj + 1 - _NB, b1)
                start_g(jj + 1, b1)
                wait_g(jj, b)
                start_s(jj, b)

        wait_g(_CH - 1, (_CH - 1) % _NB)
        start_s(_CH - 1, (_CH - 1) % _NB)
        for b in range(_NB):
            wait_s(_CH - _NB + b, (_CH - _NB + b) % _NB)
        plsc.subcore_barrier()
        pltpu.sync_copy(agg_sh.at[pl.ds(r0, _RPT)],
                        out_hbm.at[pl.ds(cid * _NP + r0, _RPT)])

    return propk(g, src3, dst3, zeros)


def _tc_phase1(x, w1, v1, degp):
    def body(x_ref, w_ref, v_ref, degp_ref, g_ref, xv_ref, dis_ref):
        deg = degp_ref[0:_N, 0:1] + degp_ref[_NP:_NP + _N, 0:1]
        dis = jnp.where(deg > 0, 1.0 / jnp.sqrt(jnp.maximum(deg, 1e-12)), 0.0)
        xv = x_ref[...]
        g_ref[...] = dis * jnp.dot(xv, w_ref[...], preferred_element_type=jnp.float32)
        xv_ref[...] = jnp.dot(xv, v_ref[...], preferred_element_type=jnp.float32)
        dis_ref[...] = dis

    return pl.pallas_call(
        body,
        out_shape=(
            jax.ShapeDtypeStruct((_N, _H), jnp.float32),
            jax.ShapeDtypeStruct((_N, _H), jnp.float32),
            jax.ShapeDtypeStruct((_N, 1), jnp.float32),
        ),
    )(x, w1, v1, degp)


def _tc_phase2(s1p, xv1, dis, b1, w2p, v2p):
    def body(s_ref, xv_ref, dis_ref, b_ref, w_ref, v_ref, g_ref, xv2_ref):
        dis = dis_ref[...]
        s = s_ref[0:_N] + s_ref[_NP:_NP + _N]
        x1 = jnp.maximum(dis * s + xv_ref[...] + b_ref[...], 0.0)
        g_ref[...] = dis * jnp.dot(x1, w_ref[...], preferred_element_type=jnp.float32)
        xv2_ref[...] = jnp.dot(x1, v_ref[...], preferred_element_type=jnp.float32)

    return pl.pallas_call(
        body,
        out_shape=(
            jax.ShapeDtypeStruct((_N, _CP), jnp.float32),
            jax.ShapeDtypeStruct((_N, _CP), jnp.float32),
        ),
    )(s1p, xv1, dis, b1, w2p, v2p)


def _tc_phase3(s2p, xv2, dis, b2p):
    def body(s_ref, xv_ref, dis_ref, b_ref, out_ref):
        dis = dis_ref[...]
        z = jnp.maximum(dis * (s_ref[0:_N] + s_ref[_NP:_NP + _N])
                        + xv_ref[...] + b_ref[...], 0.0)
        col = lax.broadcasted_iota(jnp.int32, (_N, _CP), 1)
        valid = col < _C
        zm = jnp.where(valid, z, -jnp.inf)
        m = jnp.max(zm, axis=1, keepdims=True)
        e = jnp.where(valid, jnp.exp(z - m), 0.0)
        lse = jnp.log(jnp.sum(e, axis=1, keepdims=True))
        out_ref[...] = z - m - lse

    return pl.pallas_call(
        body,
        out_shape=jax.ShapeDtypeStruct((_N, _CP), jnp.float32),
    )(s2p, xv2, dis, b2p)


def kernel(x, edge_index, W1, V1, b1, W2, V2, b2):
    src3 = edge_index[0].reshape(_NW, _CH, _K)
    dst3 = edge_index[1].reshape(_NW, _CH, _K)
    ones = jnp.ones((_K, _DEGW), jnp.float32)
    zeros_d = jnp.zeros((_NP, _DEGW), jnp.float32)
    zeros_h = jnp.zeros((_NP, _H), jnp.float32)
    zeros_c = jnp.zeros((_NP, _CP), jnp.float32)

    degp = _sc_histogram(dst3, ones, zeros_d)
    g1, xv1, dis = _tc_phase1(x, W1, V1, degp)
    s1p = _sc_prop(g1, src3, dst3, zeros_h, _H)
    w2p = jnp.pad(W2, ((0, 0), (0, _CP - _C)))
    v2p = jnp.pad(V2, ((0, 0), (0, _CP - _C)))
    g2, xv2 = _tc_phase2(s1p, xv1, dis, b1.reshape(1, _H), w2p, v2p)
    s2p = _sc_prop(g2, src3, dst3, zeros_c, _CP)
    out16 = _tc_phase3(s2p, xv2, dis, jnp.pad(b2, (0, _CP - _C)).reshape(1, _CP))
    return out16[:, :_C]


# speedup vs baseline: 39.4703x; 1.1103x over previous
import functools

import jax
import jax.numpy as jnp
from jax import lax
from jax.experimental import pallas as pl
from jax.experimental.pallas import tpu as pltpu
from jax.experimental.pallas import tpu_sc as plsc

_N = 10000
_E = 320000
_F = 128
_H = 32
_C = 10
_CP = 16

_NP = 10240
_NC = 2
_NS = 16
_NW = _NC * _NS
_EW = _E // _NW
_K = 128
_CF = _EW // _K
_KT = _EW - _CF * _K
_RPT = _NP // _NS
_DEGW = 8
_NB = 6


def _sc_mesh():
    return plsc.VectorSubcoreMesh(
        core_axis_name="c", subcore_axis_name="s",
        num_cores=_NC, num_subcores=_NS)


def _sc_histogram(dst, ones, zeros):
    @functools.partial(
        pl.kernel,
        out_type=jax.ShapeDtypeStruct((_NC * _NP, _DEGW), jnp.float32),
        mesh=_sc_mesh(),
        compiler_params=pltpu.CompilerParams(use_tc_tiling_on_sc=False),
        scratch_types=[
            pltpu.VMEM((_EW,), jnp.int32),
            pltpu.VMEM((_K, _DEGW), jnp.float32),
            pltpu.VMEM_SHARED((_NP, _DEGW), jnp.float32),
        ] + [pltpu.SemaphoreType.DMA] * _NB,
    )
    def histk(dst_hbm, ones_hbm, zeros_hbm, out_hbm, idx_v, ones_v, deg_sh,
              *sems):
        cid = lax.axis_index("c")
        sid = lax.axis_index("s")
        wid = cid * _NS + sid
        r0 = sid * _RPT
        pltpu.sync_copy(zeros_hbm.at[pl.ds(r0, _RPT)], deg_sh.at[pl.ds(r0, _RPT)])
        pltpu.sync_copy(dst_hbm.at[pl.ds(wid * _EW, _EW)], idx_v)
        pltpu.sync_copy(ones_hbm, ones_v)
        plsc.subcore_barrier()

        def idx(jj, n=_K):
            return idx_v.at[pl.ds(jj * _K, n)]

        def start_s(jj, b):
            pltpu.async_copy(ones_v, deg_sh.at[idx(jj)], sems[b], add=True)

        def wait_s(jj, b):
            pltpu.make_async_copy(ones_v, deg_sh.at[idx(jj)], sems[b]).wait()

        for b in range(_NB):
            start_s(b, b)

        @pl.loop(1, _CF // _NB)
        def _grp(g):
            for b in range(_NB):
                wait_s((g - 1) * _NB + b, b)
                start_s(g * _NB + b, b)

        for b in range(_NB):
            wait_s(_CF - _NB + b, b)
        pltpu.sync_copy(ones_v.at[pl.ds(0, _KT)],
                        deg_sh.at[idx(_CF, _KT)], add=True)
        plsc.subcore_barrier()
        pltpu.sync_copy(deg_sh.at[pl.ds(r0, _RPT)],
                        out_hbm.at[pl.ds(cid * _NP + r0, _RPT)])

    return histk(dst, ones, zeros)


def _sc_prop(g, src, dst, zeros, d):
    @functools.partial(
        pl.kernel,
        out_type=jax.ShapeDtypeStruct((_NC * _NP, d), jnp.float32),
        mesh=_sc_mesh(),
        compiler_params=pltpu.CompilerParams(use_tc_tiling_on_sc=False),
        scratch_types=[
            pltpu.VMEM((_EW,), jnp.int32),
            pltpu.VMEM((_EW,), jnp.int32),
            pltpu.VMEM((_NB, _K, d), jnp.float32),
            pltpu.VMEM_SHARED((_NP, d), jnp.float32),
        ] + [pltpu.SemaphoreType.DMA] * (2 * _NB + 1),
    )
    def propk(g_hbm, src_hbm, dst_hbm, zeros_hbm, out_hbm,
              src_v, dst_v, rows_v, agg_sh, *sems):
        sem_g = sems[:_NB]
        sem_s = sems[_NB:2 * _NB]
        sem_t = sems[2 * _NB]
        cid = lax.axis_index("c")
        sid = lax.axis_index("s")
        wid = cid * _NS + sid
        r0 = sid * _RPT
        pltpu.sync_copy(zeros_hbm.at[pl.ds(r0, _RPT)], agg_sh.at[pl.ds(r0, _RPT)])
        pltpu.sync_copy(src_hbm.at[pl.ds(wid * _EW, _EW)], src_v)
        pltpu.sync_copy(dst_hbm.at[pl.ds(wid * _EW, _EW)], dst_v)
        plsc.subcore_barrier()

        def sidx(jj, n=_K):
            return src_v.at[pl.ds(jj * _K, n)]

        def didx(jj, n=_K):
            return dst_v.at[pl.ds(jj * _K, n)]

        def start_g(jj, b):
            pltpu.async_copy(g_hbm.at[sidx(jj)], rows_v.at[b], sem_g[b])

        def wait_g(jj, b):
            pltpu.make_async_copy(g_hbm.at[sidx(jj)], rows_v.at[b],
                                  sem_g[b]).wait()

        def start_s(jj, b):
            pltpu.async_copy(rows_v.at[b], agg_sh.at[didx(jj)], sem_s[b],
                             add=True)

        def wait_s(jj, b):
            pltpu.make_async_copy(rows_v.at[b], agg_sh.at[didx(jj)],
                                  sem_s[b]).wait()

        start_g(0, 0)
        for jj in range(_NB - 1):
            b, b1 = jj % _NB, (jj + 1) % _NB
            wait_g(jj, b)
            start_s(jj, b)
            start_g(jj + 1, b1)

        @pl.loop(0, (_CF - _NB) // _NB)
        def _grp(grp):
            for k in range(_NB):
                jj = (_NB - 1) + grp * _NB + k
                b = (_NB - 1 + k) % _NB
                b1 = (b + 1) % _NB
                wait_s(jj + 1 - _NB, b1)
                start_g(jj + 1, b1)
                wait_g(jj, b)
                start_s(jj, b)

        bl = (_CF - 1) % _NB
        wait_g(_CF - 1, bl)
        start_s(_CF - 1, bl)
        wait_s(_CF - _NB, (_CF - _NB) % _NB)
        pltpu.async_copy(g_hbm.at[sidx(_CF, _KT)],
                         rows_v.at[0, pl.ds(0, _KT)], sem_t)
        for b in range(1, _NB):
            wait_s(_CF - _NB + b, (_CF - _NB + b) % _NB)
        pltpu.make_async_copy(g_hbm.at[sidx(_CF, _KT)],
                              rows_v.at[0, pl.ds(0, _KT)], sem_t).wait()
        pltpu.sync_copy(rows_v.at[0, pl.ds(0, _KT)],
                        agg_sh.at[didx(_CF, _KT)], add=True)
        plsc.subcore_barrier()
        pltpu.sync_copy(agg_sh.at[pl.ds(r0, _RPT)],
                        out_hbm.at[pl.ds(cid * _NP + r0, _RPT)])

    return propk(g, src, dst, zeros)


def _tc_phase1(x, w1, v1, degp):
    def body(x_ref, w_ref, v_ref, degp_ref, g_ref, xv_ref, dis_ref):
        deg = degp_ref[0:_N, 0:1] + degp_ref[_NP:_NP + _N, 0:1]
        dis = jnp.where(deg > 0, 1.0 / jnp.sqrt(jnp.maximum(deg, 1e-12)), 0.0)
        xv = x_ref[...]
        g_ref[...] = dis * jnp.dot(xv, w_ref[...], preferred_element_type=jnp.float32)
        xv_ref[...] = jnp.dot(xv, v_ref[...], preferred_element_type=jnp.float32)
        dis_ref[...] = dis

    return pl.pallas_call(
        body,
        out_shape=(
            jax.ShapeDtypeStruct((_N, _H), jnp.float32),
            jax.ShapeDtypeStruct((_N, _H), jnp.float32),
            jax.ShapeDtypeStruct((_N, 1), jnp.float32),
        ),
    )(x, w1, v1, degp)


def _tc_phase2(s1p, xv1, dis, b1, w2p, v2p):
    def body(s_ref, xv_ref, dis_ref, b_ref, w_ref, v_ref, g_ref, xv2_ref):
        dis = dis_ref[...]
        s = s_ref[0:_N] + s_ref[_NP:_NP + _N]
        x1 = jnp.maximum(dis * s + xv_ref[...] + b_ref[...], 0.0)
        g_ref[...] = dis * jnp.dot(x1, w_ref[...], preferred_element_type=jnp.float32)
        xv2_ref[...] = jnp.dot(x1, v_ref[...], preferred_element_type=jnp.float32)

    return pl.pallas_call(
        body,
        out_shape=(
            jax.ShapeDtypeStruct((_N, _CP), jnp.float32),
            jax.ShapeDtypeStruct((_N, _CP), jnp.float32),
        ),
    )(s1p, xv1, dis, b1, w2p, v2p)


def _tc_phase3(s2p, xv2, dis, b2p):
    def body(s_ref, xv_ref, dis_ref, b_ref, out_ref):
        dis = dis_ref[...]
        z = jnp.maximum(dis * (s_ref[0:_N] + s_ref[_NP:_NP + _N])
                        + xv_ref[...] + b_ref[...], 0.0)
        col = lax.broadcasted_iota(jnp.int32, (_N, _CP), 1)
        valid = col < _C
        zm = jnp.where(valid, z, -jnp.inf)
        m = jnp.max(zm, axis=1, keepdims=True)
        e = jnp.where(valid, jnp.exp(z - m), 0.0)
        lse = jnp.log(jnp.sum(e, axis=1, keepdims=True))
        out_ref[...] = z - m - lse

    return pl.pallas_call(
        body,
        out_shape=jax.ShapeDtypeStruct((_N, _CP), jnp.float32),
    )(s2p, xv2, dis, b2p)


def kernel(x, edge_index, W1, V1, b1, W2, V2, b2):
    src = edge_index[0]
    dst = edge_index[1]
    ones = jnp.ones((_K, _DEGW), jnp.float32)
    zeros_d = jnp.zeros((_NP, _DEGW), jnp.float32)
    zeros_h = jnp.zeros((_NP, _H), jnp.float32)
    zeros_c = jnp.zeros((_NP, _CP), jnp.float32)

    degp = _sc_histogram(dst, ones, zeros_d)
    g1, xv1, dis = _tc_phase1(x, W1, V1, degp)
    s1p = _sc_prop(g1, src, dst, zeros_h, _H)
    w2p = jnp.pad(W2, ((0, 0), (0, _CP - _C)))
    v2p = jnp.pad(V2, ((0, 0), (0, _CP - _C)))
    g2, xv2 = _tc_phase2(s1p, xv1, dis, b1.reshape(1, _H), w2p, v2p)
    s2p = _sc_prop(g2, src, dst, zeros_c, _CP)
    out16 = _tc_phase3(s2p, xv2, dis, jnp.pad(b2, (0, _CP - _C)).reshape(1, _CP))
    return out16[:, :_C]


# speedup vs baseline: 42.2408x; 1.0702x over previous
import functools

import jax
import jax.numpy as jnp
from jax import lax
from jax.experimental import pallas as pl
from jax.experimental.pallas import tpu as pltpu
from jax.experimental.pallas import tpu_sc as plsc

_N = 10000
_E = 320000
_F = 128
_H = 32
_C = 10
_CP = 16

_NP = 10240
_NC = 2
_NS = 16
_NW = _NC * _NS
_EW = _E // _NW
_K = 128
_CF = _EW // _K
_KT = _EW - _CF * _K
_RPT = _NP // _NS
_DEGW = 8
_NB = 6


def _sc_mesh():
    return plsc.VectorSubcoreMesh(
        core_axis_name="c", subcore_axis_name="s",
        num_cores=_NC, num_subcores=_NS)


def _sc_histogram(ei, ones, zeros):
    @functools.partial(
        pl.kernel,
        out_type=jax.ShapeDtypeStruct((_NC * _NP,), jnp.float32),
        mesh=_sc_mesh(),
        compiler_params=pltpu.CompilerParams(use_tc_tiling_on_sc=False,
                                             needs_layout_passes=False),
        scratch_types=[
            pltpu.VMEM((_EW,), jnp.int32),
            pltpu.VMEM((_K, _DEGW), jnp.float32),
            pltpu.VMEM((_RPT, _DEGW), jnp.float32),
            pltpu.VMEM((_RPT,), jnp.float32),
            pltpu.VMEM_SHARED((_NP, _DEGW), jnp.float32),
        ] + [pltpu.SemaphoreType.DMA] * _NB,
    )
    def histk(ei_hbm, ones_hbm, zeros_hbm, out_hbm, idx_v, ones_v, bounce_v,
              col_v, deg_sh, *sems):
        cid = lax.axis_index("c")
        sid = lax.axis_index("s")
        wid = cid * _NS + sid
        r0 = sid * _RPT
        pltpu.sync_copy(zeros_hbm.at[pl.ds(r0, _RPT)], deg_sh.at[pl.ds(r0, _RPT)])
        pltpu.sync_copy(ei_hbm.at[1, pl.ds(wid * _EW, _EW)], idx_v)
        pltpu.sync_copy(ones_hbm, ones_v)
        plsc.subcore_barrier()

        def idx(jj, n=_K):
            return idx_v.at[pl.ds(jj * _K, n)]

        def start_s(jj, b):
            pltpu.async_copy(ones_v, deg_sh.at[idx(jj)], sems[b], add=True)

        def wait_s(jj, b):
            pltpu.make_async_copy(ones_v, deg_sh.at[idx(jj)], sems[b]).wait()

        for b in range(_NB):
            start_s(b, b)

        @pl.loop(1, _CF // _NB)
        def _grp(g):
            for b in range(_NB):
                wait_s((g - 1) * _NB + b, b)
                start_s(g * _NB + b, b)

        for b in range(_NB):
            wait_s(_CF - _NB + b, b)
        pltpu.sync_copy(ones_v.at[pl.ds(0, _KT)],
                        deg_sh.at[idx(_CF, _KT)], add=True)
        plsc.subcore_barrier()
        pltpu.sync_copy(deg_sh.at[pl.ds(r0, _RPT)], bounce_v)
        lanes = lax.iota(jnp.int32, 16)
        zero16 = jnp.zeros((16,), jnp.int32)

        @pl.loop(0, _RPT // 16)
        def _ext(m):
            col_v[pl.ds(m * 16, 16)] = plsc.load_gather(
                bounce_v, [m * 16 + lanes, zero16])

        pltpu.sync_copy(col_v, out_hbm.at[pl.ds(cid * _NP + r0, _RPT)])

    return histk(ei, ones, zeros)


def _sc_prop(g, ei, zeros, d):
    @functools.partial(
        pl.kernel,
        out_type=jax.ShapeDtypeStruct((_NC * _NP, d), jnp.float32),
        mesh=_sc_mesh(),
        compiler_params=pltpu.CompilerParams(use_tc_tiling_on_sc=False),
        scratch_types=[
            pltpu.VMEM((_EW,), jnp.int32),
            pltpu.VMEM((_EW,), jnp.int32),
            pltpu.VMEM((_NB, _K, d), jnp.float32),
            pltpu.VMEM_SHARED((_NP, d), jnp.float32),
        ] + [pltpu.SemaphoreType.DMA] * (2 * _NB + 1),
    )
    def propk(g_hbm, ei_hbm, zeros_hbm, out_hbm,
              src_v, dst_v, rows_v, agg_sh, *sems):
        sem_g = sems[:_NB]
        sem_s = sems[_NB:2 * _NB]
        sem_t = sems[2 * _NB]
        cid = lax.axis_index("c")
        sid = lax.axis_index("s")
        wid = cid * _NS + sid
        r0 = sid * _RPT
        pltpu.sync_copy(zeros_hbm.at[pl.ds(r0, _RPT)], agg_sh.at[pl.ds(r0, _RPT)])
        pltpu.sync_copy(ei_hbm.at[0, pl.ds(wid * _EW, _EW)], src_v)
        pltpu.sync_copy(ei_hbm.at[1, pl.ds(wid * _EW, _EW)], dst_v)
        plsc.subcore_barrier()

        def sidx(jj, n=_K):
            return src_v.at[pl.ds(jj * _K, n)]

        def didx(jj, n=_K):
            return dst_v.at[pl.ds(jj * _K, n)]

        def start_g(jj, b):
            pltpu.async_copy(g_hbm.at[sidx(jj)], rows_v.at[b], sem_g[b])

        def wait_g(jj, b):
            pltpu.make_async_copy(g_hbm.at[sidx(jj)], rows_v.at[b],
                                  sem_g[b]).wait()

        def start_s(jj, b):
            pltpu.async_copy(rows_v.at[b], agg_sh.at[didx(jj)], sem_s[b],
                             add=True)

        def wait_s(jj, b):
            pltpu.make_async_copy(rows_v.at[b], agg_sh.at[didx(jj)],
                                  sem_s[b]).wait()

        start_g(0, 0)
        for jj in range(_NB - 1):
            b, b1 = jj % _NB, (jj + 1) % _NB
            wait_g(jj, b)
            start_s(jj, b)
            start_g(jj + 1, b1)

        @pl.loop(0, (_CF - _NB) // _NB)
        def _grp(grp):
            for k in range(_NB):
                jj = (_NB - 1) + grp * _NB + k
                b = (_NB - 1 + k) % _NB
                b1 = (b + 1) % _NB
                wait_s(jj + 1 - _NB, b1)
                start_g(jj + 1, b1)
                wait_g(jj, b)
                start_s(jj, b)

        bl = (_CF - 1) % _NB
        wait_g(_CF - 1, bl)
        start_s(_CF - 1, bl)
        wait_s(_CF - _NB, (_CF - _NB) % _NB)
        pltpu.async_copy(g_hbm.at[sidx(_CF, _KT)],
                         rows_v.at[0, pl.ds(0, _KT)], sem_t)
        for b in range(1, _NB):
            wait_s(_CF - _NB + b, (_CF - _NB + b) % _NB)
        pltpu.make_async_copy(g_hbm.at[sidx(_CF, _KT)],
                              rows_v.at[0, pl.ds(0, _KT)], sem_t).wait()
        pltpu.sync_copy(rows_v.at[0, pl.ds(0, _KT)],
                        agg_sh.at[didx(_CF, _KT)], add=True)
        plsc.subcore_barrier()
        pltpu.sync_copy(agg_sh.at[pl.ds(r0, _RPT)],
                        out_hbm.at[pl.ds(cid * _NP + r0, _RPT)])

    return propk(g, ei, zeros)


def _tc_phase1(x, w1, v1, degp):
    def body(x_ref, w_ref, v_ref, degp_ref, g_ref, xv_ref, dis_ref):
        deg = degp_ref[0:_N] + degp_ref[_NP:_NP + _N]
        dis = jnp.where(deg > 0, 1.0 / jnp.sqrt(jnp.maximum(deg, 1e-12)), 0.0)
        xv = x_ref[...]
        g_ref[...] = dis * jnp.dot(xv, w_ref[...], preferred_element_type=jnp.float32)
        xv_ref[...] = jnp.dot(xv, v_ref[...], preferred_element_type=jnp.float32)
        dis_ref[...] = dis

    return pl.pallas_call(
        body,
        out_shape=(
            jax.ShapeDtypeStruct((_N, _H), jnp.float32),
            jax.ShapeDtypeStruct((_N, _H), jnp.float32),
            jax.ShapeDtypeStruct((_N, 1), jnp.float32),
        ),
    )(x, w1, v1, degp)


def _tc_phase2(s1p, xv1, dis, b1, w2p, v2p):
    def body(s_ref, xv_ref, dis_ref, b_ref, w_ref, v_ref, g_ref, xv2_ref):
        dis = dis_ref[...]
        s = s_ref[0:_N] + s_ref[_NP:_NP + _N]
        x1 = jnp.maximum(dis * s + xv_ref[...] + b_ref[...], 0.0)
        g_ref[...] = dis * jnp.dot(x1, w_ref[...], preferred_element_type=jnp.float32)
        xv2_ref[...] = jnp.dot(x1, v_ref[...], preferred_element_type=jnp.float32)

    return pl.pallas_call(
        body,
        out_shape=(
            jax.ShapeDtypeStruct((_N, _CP), jnp.float32),
            jax.ShapeDtypeStruct((_N, _CP), jnp.float32),
        ),
    )(s1p, xv1, dis, b1, w2p, v2p)


def _tc_phase3(s2p, xv2, dis, b2p):
    def body(s_ref, xv_ref, dis_ref, b_ref, out_ref):
        dis = dis_ref[...]
        z = jnp.maximum(dis * (s_ref[0:_N] + s_ref[_NP:_NP + _N])
                        + xv_ref[...] + b_ref[...], 0.0)
        col = lax.broadcasted_iota(jnp.int32, (_N, _CP), 1)
        valid = col < _C
        zm = jnp.where(valid, z, -jnp.inf)
        m = jnp.max(zm, axis=1, keepdims=True)
        e = jnp.where(valid, jnp.exp(z - m), 0.0)
        lse = jnp.log(jnp.sum(e, axis=1, keepdims=True))
        out_ref[...] = z - m - lse

    return pl.pallas_call(
        body,
        out_shape=jax.ShapeDtypeStruct((_N, _CP), jnp.float32),
    )(s2p, xv2, dis, b2p)


def kernel(x, edge_index, W1, V1, b1, W2, V2, b2):
    ones = jnp.ones((_K, _DEGW), jnp.float32)
    zeros_d = jnp.zeros((_NP, _DEGW), jnp.float32)
    zeros_h = jnp.zeros((_NP, _H), jnp.float32)
    zeros_c = jnp.zeros((_NP, _CP), jnp.float32)

    degp = _sc_histogram(edge_index, ones, zeros_d)
    g1, xv1, dis = _tc_phase1(x, W1, V1, degp.reshape(_NC * _NP, 1))
    s1p = _sc_prop(g1, edge_index, zeros_h, _H)
    w2p = jnp.pad(W2, ((0, 0), (0, _CP - _C)))
    v2p = jnp.pad(V2, ((0, 0), (0, _CP - _C)))
    g2, xv2 = _tc_phase2(s1p, xv1, dis, b1.reshape(1, _H), w2p, v2p)
    s2p = _sc_prop(g2, edge_index, zeros_c, _CP)
    out16 = _tc_phase3(s2p, xv2, dis, jnp.pad(b2, (0, _CP - _C)).reshape(1, _CP))
    return out16[:, :_C]
